# baseline (speedup 1.0000x reference)
import functools

import jax
import jax.numpy as jnp
from jax.experimental import pallas as pl
from jax.experimental.pallas import tpu as pltpu
from jax.experimental.pallas import tpu_sc as plsc

_NE = 16
_TBLK = 1024
_NC = 2
_NS = 16
_NW = _NC * _NS


def _silu(v):
    return v * jax.nn.sigmoid(v)


def _logits_body(x_ref, rw_ref, lg_ref):
    lg_ref[...] = jax.lax.dot_general(
        x_ref[...], rw_ref[...], (((1,), (1,)), ((), ())),
        preferred_element_type=jnp.float32)


def _coef_sc_body(per_w, lg_hbm, coef_hbm, lg_v, coef_v):
    wid = jax.lax.axis_index("s") * _NC + jax.lax.axis_index("c")
    base = wid * per_w
    pltpu.sync_copy(lg_hbm.at[pl.ds(base, per_w)], lg_v)

    iota = jax.lax.iota(jnp.int32, 16)
    ninf = jnp.full((16,), -jnp.inf, jnp.float32)
    zeros = jnp.zeros((16,), jnp.float32)

    def body(gi):
        tok = gi * 16 + iota
        m1, m2 = ninf, ninf
        i1 = jnp.zeros((16,), jnp.int32)
        i2 = jnp.zeros((16,), jnp.int32)
        for e in range(_NE):
            v = plsc.load_gather(lg_v, [tok, jnp.full((16,), e, jnp.int32)])
            new_top = v > m1
            second = jnp.logical_and(v > m2, jnp.logical_not(new_top))
            m2 = jnp.where(new_top, m1, jnp.where(second, v, m2))
            i2 = jnp.where(new_top, i1, jnp.where(second, e, i2))
            m1 = jnp.where(new_top, v, m1)
            i1 = jnp.where(new_top, e, i1)
        g1 = 1.0 / (1.0 + jnp.exp(m2 - m1))
        for j in range(16):
            coef_v[gi * 16 + j, :] = zeros
        plsc.store_scatter(coef_v, [tok, i1], g1)
        plsc.store_scatter(coef_v, [tok, i2], 1.0 - g1)

    for gi in range(per_w // 16):
        body(gi)
    pltpu.sync_copy(coef_v, coef_hbm.at[pl.ds(base, per_w)])


def _moe_body(nblk, x_ref, coef_ref, w1_ref, w3_ref, w2_ref, out_ref,
              s_ref, w1b_ref, w3b_ref):
    i = pl.program_id(0)

    @pl.when(i == 0)
    def _init():
        s_ref[...] = jnp.zeros_like(s_ref)
        w1b_ref[...] = w1_ref[...].astype(jnp.bfloat16)
        w3b_ref[...] = w3_ref[...].astype(jnp.bfloat16)

    xb = x_ref[...]
    coef = coef_ref[...]

    xbf = xb.astype(jnp.bfloat16)
    a = jax.lax.dot_general(xbf, w1b_ref[...], (((1,), (1,)), ((), ())),
                            preferred_element_type=jnp.float32)
    a = _silu(a).astype(jnp.bfloat16)
    h = jax.lax.dot_general(a, w3b_ref[...], (((1,), (1,)), ((), ())),
                            preferred_element_type=jnp.float32)
    g = _silu(h)

    s_ref[...] += jax.lax.dot_general(coef, g, (((0,), (0,)), ((), ())),
                                      preferred_element_type=jnp.float32)

    out_ref[...] = jnp.zeros_like(out_ref)

    @pl.when(i == nblk - 1)
    def _final():
        rows = jax.lax.dot_general(s_ref[...], w2_ref[...],
                                   (((1,), (1,)), ((), ())),
                                   preferred_element_type=jnp.float32)
        out_ref[0:_NE, :] = rows


def kernel(x, w1, w2, w3, router_w):
    b, s, d = x.shape
    xf = x.reshape(-1, d)
    n_tok = xf.shape[0]
    nblk = n_tok // _TBLK
    per_w = n_tok // _NW

    lg = pl.pallas_call(
        _logits_body,
        grid=(nblk,),
        in_specs=[
            pl.BlockSpec((_TBLK, d), lambda i: (i, 0)),
            pl.BlockSpec((_NE, d), lambda i: (0, 0)),
        ],
        out_specs=pl.BlockSpec((_TBLK, _NE), lambda i: (i, 0)),
        out_shape=jax.ShapeDtypeStruct((n_tok, _NE), jnp.float32),
    )(xf, router_w)

    coef = pl.kernel(
        functools.partial(_coef_sc_body, per_w),
        out_type=jax.ShapeDtypeStruct((n_tok, _NE), jnp.float32),
        mesh=plsc.VectorSubcoreMesh(core_axis_name="c", subcore_axis_name="s"),
        compiler_params=pltpu.CompilerParams(needs_layout_passes=False),
        scratch_types=[
            pltpu.VMEM((per_w, _NE), jnp.float32),
            pltpu.VMEM((per_w, _NE), jnp.float32),
        ],
    )(lg)

    out = pl.pallas_call(
        functools.partial(_moe_body, nblk),
        grid=(nblk,),
        in_specs=[
            pl.BlockSpec((_TBLK, d), lambda i: ((i + 1) % nblk, 0)),
            pl.BlockSpec((_TBLK, _NE), lambda i: ((i + 1) % nblk, 0)),
            pl.BlockSpec((d, d), lambda i: (0, 0)),
            pl.BlockSpec((d, d), lambda i: (0, 0)),
            pl.BlockSpec((d, d), lambda i: (0, 0)),
        ],
        out_specs=pl.BlockSpec((_TBLK, d), lambda i: ((i + 1) % nblk, 0)),
        out_shape=jax.ShapeDtypeStruct((n_tok, d), jnp.float32),
        scratch_shapes=[
            pltpu.VMEM((_NE, d), jnp.float32),
            pltpu.VMEM((d, d), jnp.bfloat16),
            pltpu.VMEM((d, d), jnp.bfloat16),
        ],
        compiler_params=pltpu.CompilerParams(
            dimension_semantics=("arbitrary",)),
    )(xf, coef, w1, w3, w2)
    return out.reshape(b, s, d)


# speedup vs baseline: 1.0231x; 1.0231x over previous
import functools

import jax
import jax.numpy as jnp
from jax.experimental import pallas as pl
from jax.experimental.pallas import tpu as pltpu
from jax.experimental.pallas import tpu_sc as plsc

_NE = 16
_TBLK = 1024
_NC = 2
_NS = 16
_NW = _NC * _NS


def _silu(v):
    return v * jax.nn.sigmoid(v)


def _logits_body(x_ref, rw_ref, lg_ref):
    lg_ref[...] = jax.lax.dot_general(
        x_ref[...], rw_ref[...], (((1,), (1,)), ((), ())),
        preferred_element_type=jnp.float32)


def _coef_sc_body(per_w, lg_hbm, coef_hbm, lg_v, coef_v):
    wid = jax.lax.axis_index("s") * _NC + jax.lax.axis_index("c")
    base = wid * per_w
    pltpu.sync_copy(lg_hbm.at[pl.ds(base, per_w)], lg_v)

    iota = jax.lax.iota(jnp.int32, 16)
    ninf = jnp.full((16,), -jnp.inf, jnp.float32)
    zeros = jnp.zeros((16,), jnp.float32)

    def body(gi, carry):
        tok = gi * 16 + iota
        m1, m2 = ninf, ninf
        i1 = jnp.zeros((16,), jnp.int32)
        i2 = jnp.zeros((16,), jnp.int32)
        for e in range(_NE):
            v = plsc.load_gather(lg_v, [tok, jnp.full((16,), e, jnp.int32)])
            new_top = v > m1
            second = jnp.logical_and(v > m2, jnp.logical_not(new_top))
            m2 = jnp.where(new_top, m1, jnp.where(second, v, m2))
            i2 = jnp.where(new_top, i1, jnp.where(second, e, i2))
            m1 = jnp.where(new_top, v, m1)
            i1 = jnp.where(new_top, e, i1)
        g1 = 1.0 / (1.0 + jnp.exp(m2 - m1))
        for j in range(16):
            coef_v[gi * 16 + j, :] = zeros
        plsc.store_scatter(coef_v, [tok, i1], g1)
        plsc.store_scatter(coef_v, [tok, i2], 1.0 - g1)
        return carry

    jax.lax.fori_loop(0, per_w // 16, body, 0)
    pltpu.sync_copy(coef_v, coef_hbm.at[pl.ds(base, per_w)])


def _moe_body(nblk, x_ref, coef_ref, w1_ref, w3_ref, w2_ref, out_ref,
              s_ref, w1b_ref, w3b_ref):
    i = pl.program_id(0)

    @pl.when(i == 0)
    def _init():
        s_ref[...] = jnp.zeros_like(s_ref)
        w1b_ref[...] = w1_ref[...].astype(jnp.bfloat16)
        w3b_ref[...] = w3_ref[...].astype(jnp.bfloat16)

    xb = x_ref[...]
    coef = coef_ref[...]

    xbf = xb.astype(jnp.bfloat16)
    a = jax.lax.dot_general(xbf, w1b_ref[...], (((1,), (1,)), ((), ())),
                            preferred_element_type=jnp.float32)
    a = _silu(a).astype(jnp.bfloat16)
    h = jax.lax.dot_general(a, w3b_ref[...], (((1,), (1,)), ((), ())),
                            preferred_element_type=jnp.float32)
    g = _silu(h)

    s_ref[...] += jax.lax.dot_general(coef, g, (((0,), (0,)), ((), ())),
                                      preferred_element_type=jnp.float32)

    out_ref[...] = jnp.zeros_like(out_ref)

    @pl.when(i == nblk - 1)
    def _final():
        rows = jax.lax.dot_general(s_ref[...], w2_ref[...],
                                   (((1,), (1,)), ((), ())),
                                   preferred_element_type=jnp.float32)
        out_ref[0:_NE, :] = rows


def kernel(x, w1, w2, w3, router_w):
    b, s, d = x.shape
    xf = x.reshape(-1, d)
    n_tok = xf.shape[0]
    nblk = n_tok // _TBLK
    per_w = n_tok // _NW

    lg = pl.pallas_call(
        _logits_body,
        grid=(nblk,),
        in_specs=[
            pl.BlockSpec((_TBLK, d), lambda i: (i, 0)),
            pl.BlockSpec((_NE, d), lambda i: (0, 0)),
        ],
        out_specs=pl.BlockSpec((_TBLK, _NE), lambda i: (i, 0)),
        out_shape=jax.ShapeDtypeStruct((n_tok, _NE), jnp.float32),
    )(xf, router_w)

    coef = pl.kernel(
        functools.partial(_coef_sc_body, per_w),
        out_type=jax.ShapeDtypeStruct((n_tok, _NE), jnp.float32),
        mesh=plsc.VectorSubcoreMesh(core_axis_name="c", subcore_axis_name="s"),
        compiler_params=pltpu.CompilerParams(needs_layout_passes=False),
        scratch_types=[
            pltpu.VMEM((per_w, _NE), jnp.float32),
            pltpu.VMEM((per_w, _NE), jnp.float32),
        ],
    )(lg)

    out = pl.pallas_call(
        functools.partial(_moe_body, nblk),
        grid=(nblk,),
        in_specs=[
            pl.BlockSpec((_TBLK, d), lambda i: ((i + 1) % nblk, 0)),
            pl.BlockSpec((_TBLK, _NE), lambda i: ((i + 1) % nblk, 0)),
            pl.BlockSpec((d, d), lambda i: (0, 0)),
            pl.BlockSpec((d, d), lambda i: (0, 0)),
            pl.BlockSpec((d, d), lambda i: (0, 0)),
        ],
        out_specs=pl.BlockSpec((_TBLK, d), lambda i: ((i + 1) % nblk, 0)),
        out_shape=jax.ShapeDtypeStruct((n_tok, d), jnp.float32),
        scratch_shapes=[
            pltpu.VMEM((_NE, d), jnp.float32),
            pltpu.VMEM((d, d), jnp.bfloat16),
            pltpu.VMEM((d, d), jnp.bfloat16),
        ],
        compiler_params=pltpu.CompilerParams(
            dimension_semantics=("arbitrary",)),
    )(xf, coef, w1, w3, w2)
    return out.reshape(b, s, d)
